# baseline (speedup 1.0000x reference)
import functools

import jax
import jax.numpy as jnp
from jax import lax
from jax.experimental import pallas as pl
from jax.experimental.pallas import tpu as pltpu
from jax.experimental.pallas import tpu_sc as plsc

D_MODEL = 64
D_VOCAB = 1000000
D_PAD = 128


_TBLK = 16384


_SUB = 128


def _transpose_body(w_ref, out_ref):
    w = w_ref[...]
    r = lax.broadcasted_iota(jnp.int32, (_SUB, _SUB), 0)
    c = lax.broadcasted_iota(jnp.int32, (_SUB, _SUB), 1)
    eye = (r == c).astype(jnp.float32)
    zeros = jnp.zeros((_TBLK, D_PAD - D_MODEL), jnp.float32)
    parts = []
    for j in range(_TBLK // _SUB):
        sub = w[:, j * _SUB:(j + 1) * _SUB]
        parts.append(lax.dot_general(
            eye, sub, (((1,), (1,)), ((), ())),
            precision=lax.Precision.HIGHEST,
            preferred_element_type=jnp.float32))
    out_ref[...] = jnp.concatenate(
        [jnp.concatenate(parts, axis=0), zeros], axis=1)


def _transpose_table(W_E):
    return pl.pallas_call(
        _transpose_body,
        grid=(pl.cdiv(D_VOCAB, _TBLK),),
        in_specs=[pl.BlockSpec((D_MODEL, _TBLK), lambda i: (0, i))],
        out_specs=pl.BlockSpec((_TBLK, D_PAD), lambda i: (i, 0)),
        out_shape=jax.ShapeDtypeStruct((D_VOCAB, D_PAD), jnp.float32),
    )(W_E)


_BB = 128
_NP = 200
_NBT = 32


def _make_gather():
    info = plsc.get_sparse_core_info()
    NW = info.num_cores * info.num_subcores
    n_units = _NP * _NBT
    u_per_w = n_units // NW
    idx_per_w = u_per_w * _BB
    mesh = plsc.VectorSubcoreMesh(core_axis_name="c", subcore_axis_name="s")

    @functools.partial(
        pl.kernel,
        mesh=mesh,
        compiler_params=pltpu.CompilerParams(use_tc_tiling_on_sc=True,
                                             needs_layout_passes=False),
        out_type=jax.ShapeDtypeStruct((_NP, 8, _NBT, 8, _BB), jnp.float32),
        scratch_types=[
            pltpu.VMEM((idx_per_w,), jnp.int32),
            pltpu.VMEM((_BB, D_PAD), jnp.float32),
            pltpu.VMEM((_BB, D_PAD), jnp.float32),
            pltpu.VMEM((8, 8, _BB), jnp.float32),
            pltpu.VMEM((8, 8, _BB), jnp.float32),
            pltpu.SemaphoreType.DMA,
            pltpu.SemaphoreType.DMA,
            pltpu.SemaphoreType.DMA,
            pltpu.SemaphoreType.DMA,
        ],
    )
    def gather_kernel(table_hbm, idx_hbm, out_hbm, idx_v, rows0, rows1,
                      t0, t1, gsem0, gsem1, wsem0, wsem1):
        wid = lax.axis_index("s") * info.num_cores + lax.axis_index("c")
        ubase = wid * u_per_w
        pltpu.sync_copy(idx_hbm.at[pl.ds(ubase * _BB, idx_per_w)], idx_v)

        def gstart(u_local, rows, gsem):
            pltpu.async_copy(
                table_hbm.at[idx_v.at[pl.ds(u_local * _BB, _BB)]], rows, gsem)

        def transpose_unit(rows, t):
            def dt_body(dt, carry):
                for di in range(8):
                    col = jnp.full((16,), dt * 8 + di, jnp.int32)
                    for k in range(8):
                        rowi = lax.iota(jnp.int32, 16) + 16 * k
                        t[dt, di, pl.ds(16 * k, 16)] = plsc.load_gather(
                            rows, [rowi, col])
                return carry

            lax.fori_loop(0, 8, dt_body, 0)

        def wdst(u_local):
            u = ubase + u_local
            return out_hbm.at[u // _NBT, :, u % _NBT]

        def step(c, phase, rows, t, gsem, wsem):
            u_local = 2 * c + phase
            pltpu.make_async_copy(table_hbm.at[idx_v.at[pl.ds(0, _BB)]],
                                  rows, gsem).wait()

            @pl.when(c > 0)
            def _():
                pltpu.make_async_copy(t, wdst(0), wsem).wait()

            transpose_unit(rows, t)

            @pl.when(u_local + 2 < u_per_w)
            def _():
                gstart(u_local + 2, rows, gsem)

            pltpu.async_copy(t, wdst(u_local), wsem)

        gstart(0, rows0, gsem0)
        gstart(1, rows1, gsem1)

        def body(c, carry):
            step(c, 0, rows0, t0, gsem0, wsem0)
            step(c, 1, rows1, t1, gsem1, wsem1)
            return carry

        lax.fori_loop(0, u_per_w // 2, body, 0)
        pltpu.make_async_copy(t0, wdst(0), wsem0).wait()
        pltpu.make_async_copy(t1, wdst(0), wsem1).wait()

    return gather_kernel


def kernel(x, W_E):
    b, p = x.shape
    W_T = _transpose_table(W_E)
    idx = jnp.swapaxes(x, 0, 1).reshape(-1).astype(jnp.int32)
    out_phys = _make_gather()(W_T, idx)
    return out_phys.transpose(2, 4, 0, 1, 3).reshape(b, p, D_MODEL)


# speedup vs baseline: 1.1648x; 1.1648x over previous
import functools

import jax
import jax.numpy as jnp
from jax import lax
from jax.experimental import pallas as pl
from jax.experimental.pallas import tpu as pltpu
from jax.experimental.pallas import tpu_sc as plsc

D_MODEL = 64
D_VOCAB = 1000000
D_PAD = 128


_TBLK = 16384


_SUB = 128


def _transpose_body(w_ref, out_ref):
    w = w_ref[...]
    r = lax.broadcasted_iota(jnp.int32, (_SUB, _SUB), 0)
    c = lax.broadcasted_iota(jnp.int32, (_SUB, _SUB), 1)
    eye = (r == c).astype(jnp.float32)
    zeros = jnp.zeros((_TBLK, D_PAD - D_MODEL), jnp.float32)
    parts = []
    for j in range(_TBLK // _SUB):
        sub = w[:, j * _SUB:(j + 1) * _SUB]
        parts.append(lax.dot_general(
            eye, sub, (((1,), (1,)), ((), ())),
            precision=lax.Precision.HIGHEST,
            preferred_element_type=jnp.float32))
    out_ref[...] = jnp.concatenate(
        [jnp.concatenate(parts, axis=0), zeros], axis=1)


def _transpose_table(W_E):
    return pl.pallas_call(
        _transpose_body,
        grid=(pl.cdiv(D_VOCAB, _TBLK),),
        in_specs=[pl.BlockSpec((D_MODEL, _TBLK), lambda i: (0, i))],
        out_specs=pl.BlockSpec((_TBLK, D_PAD), lambda i: (i, 0)),
        out_shape=jax.ShapeDtypeStruct((D_VOCAB, D_PAD), jnp.float32),
    )(W_E)


_BB = 128
_NP = 200
_NBT = 32


def _make_gather():
    info = plsc.get_sparse_core_info()
    NW = info.num_cores * info.num_subcores
    n_units = _NP * _NBT
    u_per_w = n_units // NW
    idx_per_w = u_per_w * _BB
    mesh = plsc.VectorSubcoreMesh(core_axis_name="c", subcore_axis_name="s")

    @functools.partial(
        pl.kernel,
        mesh=mesh,
        compiler_params=pltpu.CompilerParams(use_tc_tiling_on_sc=True,
                                             needs_layout_passes=False),
        out_type=jax.ShapeDtypeStruct((_NP, 8, _NBT, 8, _BB), jnp.float32),
        scratch_types=[
            pltpu.VMEM((idx_per_w,), jnp.int32),
            pltpu.VMEM((_BB, D_PAD), jnp.float32),
            pltpu.VMEM((_BB, D_PAD), jnp.float32),
            pltpu.VMEM((8, 8, _BB + 1), jnp.float32),
            pltpu.VMEM((8, 8, _BB + 1), jnp.float32),
            pltpu.SemaphoreType.DMA,
            pltpu.SemaphoreType.DMA,
            pltpu.SemaphoreType.DMA,
            pltpu.SemaphoreType.DMA,
        ],
    )
    def gather_kernel(table_hbm, idx_hbm, out_hbm, idx_v, rows0, rows1,
                      t0, t1, gsem0, gsem1, wsem0, wsem1):
        wid = lax.axis_index("s") * info.num_cores + lax.axis_index("c")
        ubase = wid * u_per_w
        pltpu.sync_copy(idx_hbm.at[pl.ds(ubase * _BB, idx_per_w)], idx_v)

        def gstart(u_local, rows, gsem):
            pltpu.async_copy(
                table_hbm.at[idx_v.at[pl.ds(u_local * _BB, _BB)]], rows, gsem)

        def transpose_unit(rows, t):
            def b_body(b, carry):
                col = jnp.full((16,), b, jnp.int32)
                for k in range(D_MODEL // 16):
                    d = lax.iota(jnp.int32, 16) + 16 * k
                    v = rows[b, pl.ds(16 * k, 16)]
                    plsc.store_scatter(t, [d // 8, d % 8, col], v)
                return carry

            lax.fori_loop(0, _BB, b_body, 0)

        def wdst(u_local):
            u = ubase + u_local
            return out_hbm.at[u // _NBT, :, u % _NBT]

        def step(c, phase, rows, t, gsem, wsem):
            u_local = 2 * c + phase
            pltpu.make_async_copy(table_hbm.at[idx_v.at[pl.ds(0, _BB)]],
                                  rows, gsem).wait()

            @pl.when(c > 0)
            def _():
                pltpu.make_async_copy(
                    t.at[:, :, pl.ds(0, _BB)], wdst(0), wsem).wait()

            transpose_unit(rows, t)

            @pl.when(u_local + 2 < u_per_w)
            def _():
                gstart(u_local + 2, rows, gsem)

            pltpu.async_copy(t.at[:, :, pl.ds(0, _BB)], wdst(u_local), wsem)

        gstart(0, rows0, gsem0)
        gstart(1, rows1, gsem1)

        def body(c, carry):
            step(c, 0, rows0, t0, gsem0, wsem0)
            step(c, 1, rows1, t1, gsem1, wsem1)
            return carry

        lax.fori_loop(0, u_per_w // 2, body, 0)
        pltpu.make_async_copy(t0.at[:, :, pl.ds(0, _BB)], wdst(0), wsem0).wait()
        pltpu.make_async_copy(t1.at[:, :, pl.ds(0, _BB)], wdst(0), wsem1).wait()

    return gather_kernel


def kernel(x, W_E):
    b, p = x.shape
    W_T = _transpose_table(W_E)
    idx = jnp.swapaxes(x, 0, 1).reshape(-1).astype(jnp.int32)
    out_phys = _make_gather()(W_T, idx)
    return out_phys.transpose(2, 4, 0, 1, 3).reshape(b, p, D_MODEL)
